# baseline (speedup 1.0000x reference)
import functools

import jax
import jax.numpy as jnp
from jax import lax
from jax.experimental import pallas as pl
from jax.experimental.pallas import tpu as pltpu
from jax.experimental.pallas import tpu_sc as plsc

_VOCAB = 8192
_DIM = 256
_BETA = 0.25

_TB = 1024
_NT = 8192 // _TB
_CH = 512
_NCH = _VOCAB // _CH

_NC, _NS = 2, 16
_NW = _NC * _NS
_BPW = 8192 // _NW


_SEG = 16
_NSEG = _CH // _SEG


def _argmin_body(f_ref, a_ref, emb_ref, idx_ref, loss_ref, w_ref, acc):
    s = pl.program_id(0)

    @pl.when(s == 0)
    def _():
        acc[...] = jnp.zeros_like(acc)
        r_io = lax.broadcasted_iota(jnp.int32, (_NSEG, _CH), 1)
        s_io = lax.broadcasted_iota(jnp.int32, (_NSEG, _CH), 0)
        j = r_io - s_io * _SEG
        jc = jnp.clip(j, 0, _SEG - 1)
        pw = lax.bitcast_convert_type((127 + 15 - jc) << 23, jnp.float32)
        w_ref[...] = jnp.where((j >= 0) & (j < _SEG), pw, 0.0)

    fb = f_ref[0]
    aa = a_ref[0]

    rlmin = None
    ridx = None

    for c in range(_NCH):
        m = lax.dot_general(
            emb_ref[c * _CH:(c + 1) * _CH, :], fb,
            (((1,), (0,)), ((), ())),
            preferred_element_type=jnp.float32,
        )
        cmax = jnp.max(m, axis=0, keepdims=True)
        clmin = aa - (cmax + cmax)
        lmin_i = lax.bitcast_convert_type(clmin, jnp.int32)
        ulp = lax.bitcast_convert_type(lmin_i + 1, jnp.float32) - clmin
        t2 = (aa - clmin) - ulp * 0.5
        tt = t2 * 0.5
        even = (lmin_i & 1) == 0
        tt_i = lax.bitcast_convert_type(tt, jnp.int32)
        step = jnp.where(tt > 0.0, jnp.int32(-1), jnp.int32(1))
        t_adj = lax.bitcast_convert_type(
            jnp.where(even, tt_i + step, tt_i), jnp.float32)
        mask = jnp.where(m > t_adj, 1.0, 0.0)
        bm = lax.dot_general(
            w_ref[...], mask,
            (((1,), (0,)), ((), ())),
            preferred_element_type=jnp.float32,
        )
        run_bm = bm[_NSEG - 1:_NSEG]
        run_s = jnp.full((1, _TB), float(_NSEG - 1), jnp.float32)
        for g in range(_NSEG - 2, -1, -1):
            bg = bm[g:g + 1]
            hit = bg > 0.0
            run_bm = jnp.where(hit, bg, run_bm)
            run_s = jnp.where(hit, float(g), run_s)
        e = (lax.bitcast_convert_type(run_bm, jnp.int32) >> 23) - 127
        jstar = (15 - e).astype(jnp.float32)
        cidx = float(c * _CH) + run_s * float(_SEG) + jstar
        if c == 0:
            rlmin, ridx = clmin, cidx
        else:
            tie = clmin == rlmin
            better = clmin < rlmin
            ridx = jnp.where(better, cidx,
                             jnp.where(tie, jnp.minimum(ridx, cidx), ridx))
            rlmin = jnp.where(better, clmin, rlmin)

    idx_ref[...] = ridx.astype(jnp.int32).reshape(1, 1, _TB)
    acc[...] = acc[...] + jnp.sum(rlmin)

    @pl.when(s == _NT - 1)
    def _():
        loss_ref[...] = acc[...] * ((1.0 + _BETA) / (8192.0 * _DIM))


def _distance_argmin(f3, a3, emb_weight):
    idx3, loss11 = pl.pallas_call(
        _argmin_body,
        grid=(_NT,),
        in_specs=[
            pl.BlockSpec((1, _DIM, _TB), lambda s: (s, 0, 0)),
            pl.BlockSpec((1, 1, _TB), lambda s: (s, 0, 0)),
            pl.BlockSpec((_VOCAB, _DIM), lambda s: (0, 0)),
        ],
        out_specs=[
            pl.BlockSpec((1, 1, _TB), lambda s: (s, 0, 0)),
            pl.BlockSpec((1, 1), lambda s: (0, 0)),
        ],
        out_shape=[
            jax.ShapeDtypeStruct((_NT, 1, _TB), jnp.int32),
            jax.ShapeDtypeStruct((1, 1), jnp.float32),
        ],
        scratch_shapes=[
            pltpu.VMEM((_NSEG, _CH), jnp.float32),
            pltpu.VMEM((1, 1), jnp.float32),
        ],
    )(f3, a3, emb_weight)
    return idx3, loss11


@functools.cache
def _make_sc_gather():
    @functools.partial(
        pl.kernel,
        mesh=plsc.VectorSubcoreMesh(core_axis_name="c", subcore_axis_name="s"),
        out_type=jax.ShapeDtypeStruct((8192, _DIM), jnp.float32),
        scratch_types=[
            pltpu.VMEM((_BPW,), jnp.int32),
            pltpu.VMEM((_BPW, _DIM), jnp.float32),
            pltpu.SemaphoreType.DMA,
        ],
    )
    def _sc_gather(table_hbm, idx_hbm, out_hbm, idx_v, rows_v, sem):
        wid = lax.axis_index("s") * _NC + lax.axis_index("c")
        base = wid * _BPW
        pltpu.sync_copy(idx_hbm.at[pl.ds(base, _BPW)], idx_v)
        pltpu.async_copy(table_hbm.at[idx_v], rows_v, sem).wait()
        pltpu.sync_copy(rows_v, out_hbm.at[pl.ds(base, _BPW)])

    return _sc_gather


def kernel(f_BChw, emb_weight):
    B, C, H, W = f_BChw.shape
    f_flat = jnp.transpose(f_BChw, (0, 2, 3, 1)).reshape(-1, C)
    a3 = jnp.sum(f_flat ** 2, axis=1).reshape(_NT, 1, _TB)
    f3 = f_BChw.reshape(_NT, C, _TB)

    idx3, loss11 = _distance_argmin(f3, a3, emb_weight)
    idx = idx3.reshape(-1)

    q_flat = _make_sc_gather()(emb_weight, idx)

    quantized_st = jnp.transpose(q_flat.reshape(B, H, W, C), (0, 3, 1, 2))
    return (quantized_st, loss11[0, 0], idx.reshape(B, H, W))


# speedup vs baseline: 1.3383x; 1.3383x over previous
import functools

import jax
import jax.numpy as jnp
from jax import lax
from jax.experimental import pallas as pl
from jax.experimental.pallas import tpu as pltpu
from jax.experimental.pallas import tpu_sc as plsc

_VOCAB = 8192
_DIM = 256
_BETA = 0.25

_TB = 1024
_NT = 8192 // _TB
_CH = 1024
_NCH = _VOCAB // _CH

_NC, _NS = 2, 16
_NW = _NC * _NS
_BPW = 8192 // _NW


def _argmin_body(f_ref, a_ref, emb_ref, idx_ref, loss_ref, iota_ref, acc):
    s = pl.program_id(0)

    @pl.when(s == 0)
    def _():
        acc[...] = jnp.zeros_like(acc)
        iota_ref[...] = lax.broadcasted_iota(
            jnp.int32, (_CH, _TB), 0).astype(jnp.float32)

    fb = f_ref[0]
    aa = a_ref[0]
    row = iota_ref[...]

    rlmin = None
    ridx = None

    for c in range(_NCH):
        m = lax.dot_general(
            emb_ref[c * _CH:(c + 1) * _CH, :], fb,
            (((1,), (0,)), ((), ())),
            preferred_element_type=jnp.float32,
        )
        cmax = jnp.max(m, axis=0, keepdims=True)
        clmin = aa - (cmax + cmax)
        lmin_i = lax.bitcast_convert_type(clmin, jnp.int32)
        ulp = lax.bitcast_convert_type(lmin_i + 1, jnp.float32) - clmin
        t2 = (aa - clmin) - ulp * 0.5
        tt = t2 * 0.5
        even = (lmin_i & 1) == 0
        tt_i = lax.bitcast_convert_type(tt, jnp.int32)
        step = jnp.where(tt > 0.0, jnp.int32(-1), jnp.int32(1))
        t_adj = lax.bitcast_convert_type(
            jnp.where(even, tt_i + step, tt_i), jnp.float32)
        cidx = jnp.min(jnp.where(m > t_adj, row, float(_CH)),
                       axis=0, keepdims=True) + float(c * _CH)
        if c == 0:
            rlmin, ridx = clmin, cidx
        else:
            tie = clmin == rlmin
            better = clmin < rlmin
            ridx = jnp.where(better, cidx,
                             jnp.where(tie, jnp.minimum(ridx, cidx), ridx))
            rlmin = jnp.where(better, clmin, rlmin)

    idx_ref[...] = ridx.astype(jnp.int32).reshape(1, 1, _TB)
    acc[...] = acc[...] + jnp.sum(rlmin)

    @pl.when(s == _NT - 1)
    def _():
        loss_ref[...] = acc[...] * ((1.0 + _BETA) / (8192.0 * _DIM))


def _distance_argmin(f3, a3, emb_weight):
    idx3, loss11 = pl.pallas_call(
        _argmin_body,
        grid=(_NT,),
        in_specs=[
            pl.BlockSpec((1, _DIM, _TB), lambda s: (s, 0, 0)),
            pl.BlockSpec((1, 1, _TB), lambda s: (s, 0, 0)),
            pl.BlockSpec((_VOCAB, _DIM), lambda s: (0, 0)),
        ],
        out_specs=[
            pl.BlockSpec((1, 1, _TB), lambda s: (s, 0, 0)),
            pl.BlockSpec((1, 1), lambda s: (0, 0)),
        ],
        out_shape=[
            jax.ShapeDtypeStruct((_NT, 1, _TB), jnp.int32),
            jax.ShapeDtypeStruct((1, 1), jnp.float32),
        ],
        scratch_shapes=[
            pltpu.VMEM((_CH, _TB), jnp.float32),
            pltpu.VMEM((1, 1), jnp.float32),
        ],
    )(f3, a3, emb_weight)
    return idx3, loss11


@functools.cache
def _make_sc_gather():
    @functools.partial(
        pl.kernel,
        mesh=plsc.VectorSubcoreMesh(core_axis_name="c", subcore_axis_name="s"),
        out_type=jax.ShapeDtypeStruct((8192, _DIM), jnp.float32),
        scratch_types=[
            pltpu.VMEM((_BPW,), jnp.int32),
            pltpu.VMEM((_BPW, _DIM), jnp.float32),
            pltpu.SemaphoreType.DMA,
        ],
    )
    def _sc_gather(table_hbm, idx_hbm, out_hbm, idx_v, rows_v, sem):
        wid = lax.axis_index("s") * _NC + lax.axis_index("c")
        base = wid * _BPW
        pltpu.sync_copy(idx_hbm.at[pl.ds(base, _BPW)], idx_v)
        pltpu.async_copy(table_hbm.at[idx_v], rows_v, sem).wait()
        pltpu.sync_copy(rows_v, out_hbm.at[pl.ds(base, _BPW)])

    return _sc_gather


def kernel(f_BChw, emb_weight):
    B, C, H, W = f_BChw.shape
    f_flat = jnp.transpose(f_BChw, (0, 2, 3, 1)).reshape(-1, C)
    a3 = jnp.sum(f_flat ** 2, axis=1).reshape(_NT, 1, _TB)
    f3 = f_BChw.reshape(_NT, C, _TB)

    idx3, loss11 = _distance_argmin(f3, a3, emb_weight)
    idx = idx3.reshape(-1)

    q_flat = _make_sc_gather()(emb_weight, idx)

    quantized_st = jnp.transpose(q_flat.reshape(B, H, W, C), (0, 3, 1, 2))
    return (quantized_st, loss11[0, 0], idx.reshape(B, H, W))


# speedup vs baseline: 1.4317x; 1.0698x over previous
import functools

import jax
import jax.numpy as jnp
from jax import lax
from jax.experimental import pallas as pl
from jax.experimental.pallas import tpu as pltpu
from jax.experimental.pallas import tpu_sc as plsc

_VOCAB = 8192
_DIM = 256
_BETA = 0.25

_TB = 1024
_NT = 8192 // _TB
_CH = 2048
_NCH = _VOCAB // _CH

_NC, _NS = 2, 16
_NW = _NC * _NS
_BPW = 8192 // _NW


def _argmin_body(f_ref, a_ref, emb_ref, idx_ref, loss_ref, iota_ref, acc):
    s = pl.program_id(0)

    @pl.when(s == 0)
    def _():
        acc[...] = jnp.zeros_like(acc)
        iota_ref[...] = lax.broadcasted_iota(
            jnp.int32, (_CH, _TB), 0).astype(jnp.float32)

    fb = f_ref[0]
    aa = a_ref[0]
    row = iota_ref[...]

    rlmin = None
    ridx = None

    for c in range(_NCH):
        m = lax.dot_general(
            emb_ref[c * _CH:(c + 1) * _CH, :], fb,
            (((1,), (0,)), ((), ())),
            preferred_element_type=jnp.float32,
        )
        cmax = jnp.max(m, axis=0, keepdims=True)
        clmin = aa - (cmax + cmax)
        lmin_i = lax.bitcast_convert_type(clmin, jnp.int32)
        ulp = lax.bitcast_convert_type(lmin_i + 1, jnp.float32) - clmin
        t2 = (aa - clmin) - ulp * 0.5
        tt = t2 * 0.5
        even = (lmin_i & 1) == 0
        tt_i = lax.bitcast_convert_type(tt, jnp.int32)
        step = jnp.where(tt > 0.0, jnp.int32(-1), jnp.int32(1))
        t_adj = lax.bitcast_convert_type(
            jnp.where(even, tt_i + step, tt_i), jnp.float32)
        cidx = jnp.min(jnp.where(m > t_adj, row, float(_CH)),
                       axis=0, keepdims=True) + float(c * _CH)
        if c == 0:
            rlmin, ridx = clmin, cidx
        else:
            tie = clmin == rlmin
            better = clmin < rlmin
            ridx = jnp.where(better, cidx,
                             jnp.where(tie, jnp.minimum(ridx, cidx), ridx))
            rlmin = jnp.where(better, clmin, rlmin)

    idx_ref[...] = ridx.astype(jnp.int32).reshape(1, 1, _TB)
    acc[...] = acc[...] + jnp.sum(rlmin)

    @pl.when(s == _NT - 1)
    def _():
        loss_ref[...] = acc[...] * ((1.0 + _BETA) / (8192.0 * _DIM))


def _distance_argmin(f3, a3, emb_weight):
    idx3, loss11 = pl.pallas_call(
        _argmin_body,
        grid=(_NT,),
        in_specs=[
            pl.BlockSpec((1, _DIM, _TB), lambda s: (s, 0, 0)),
            pl.BlockSpec((1, 1, _TB), lambda s: (s, 0, 0)),
            pl.BlockSpec((_VOCAB, _DIM), lambda s: (0, 0)),
        ],
        out_specs=[
            pl.BlockSpec((1, 1, _TB), lambda s: (s, 0, 0)),
            pl.BlockSpec((1, 1), lambda s: (0, 0)),
        ],
        out_shape=[
            jax.ShapeDtypeStruct((_NT, 1, _TB), jnp.int32),
            jax.ShapeDtypeStruct((1, 1), jnp.float32),
        ],
        scratch_shapes=[
            pltpu.VMEM((_CH, _TB), jnp.float32),
            pltpu.VMEM((1, 1), jnp.float32),
        ],
    )(f3, a3, emb_weight)
    return idx3, loss11


@functools.cache
def _make_sc_gather():
    @functools.partial(
        pl.kernel,
        mesh=plsc.VectorSubcoreMesh(core_axis_name="c", subcore_axis_name="s"),
        out_type=jax.ShapeDtypeStruct((8192, _DIM), jnp.float32),
        scratch_types=[
            pltpu.VMEM((_BPW,), jnp.int32),
            pltpu.VMEM((_BPW, _DIM), jnp.float32),
            pltpu.SemaphoreType.DMA,
        ],
    )
    def _sc_gather(table_hbm, idx_hbm, out_hbm, idx_v, rows_v, sem):
        wid = lax.axis_index("s") * _NC + lax.axis_index("c")
        base = wid * _BPW
        pltpu.sync_copy(idx_hbm.at[pl.ds(base, _BPW)], idx_v)
        pltpu.async_copy(table_hbm.at[idx_v], rows_v, sem).wait()
        pltpu.sync_copy(rows_v, out_hbm.at[pl.ds(base, _BPW)])

    return _sc_gather


def kernel(f_BChw, emb_weight):
    B, C, H, W = f_BChw.shape
    f_flat = jnp.transpose(f_BChw, (0, 2, 3, 1)).reshape(-1, C)
    a3 = jnp.sum(f_flat ** 2, axis=1).reshape(_NT, 1, _TB)
    f3 = f_BChw.reshape(_NT, C, _TB)

    idx3, loss11 = _distance_argmin(f3, a3, emb_weight)
    idx = idx3.reshape(-1)

    q_flat = _make_sc_gather()(emb_weight, idx)

    quantized_st = jnp.transpose(q_flat.reshape(B, H, W, C), (0, 3, 1, 2))
    return (quantized_st, loss11[0, 0], idx.reshape(B, H, W))
